# baseline (speedup 1.0000x reference)
import jax
import jax.numpy as jnp
from jax import lax
from jax.experimental import pallas as pl
from jax.experimental.pallas import tpu as pltpu
from jax.experimental.pallas import tpu_sc as plsc

N = 10000
E = 320000
D = 128
DH = D // 2

NC = 2
NS = 16
CPT = E // NS
CHUNK = 80
NCHUNK = CPT // CHUNK
HCHUNK = NCHUNK // 2
NP = 10240
RPT = NP // NS
ZR = 128


NB = 5


def _sc_body(xv_hbm, src_hbm, dst_hbm, out_s, out_c,
             sidx_all, didx_all, rows, ones16, zrows, z16,
             isem, gsem, ssem, csem, zsem, acc, cnt):
    c = lax.axis_index("c")
    s = lax.axis_index("s")
    jbase = s * NCHUNK
    rbase = s * RPT

    zero16 = jnp.zeros((16,), jnp.float32)
    one16 = jnp.ones((16,), jnp.float32)

    idx_in = pltpu.async_copy(
        src_hbm.at[pl.ds(jbase, NCHUNK)], sidx_all, isem)
    idx_in2 = pltpu.async_copy(
        dst_hbm.at[pl.ds(jbase, NCHUNK)], didx_all, isem)

    def fill_z(i, carry):
        for k in range(DH // 16):
            zrows[i, pl.ds(k * 16, 16)] = zero16
        z16[i, :] = zero16
        return carry

    lax.fori_loop(0, ZR, fill_z, 0)

    def fill_one(i, carry):
        ones16[i, :] = one16
        return carry

    lax.fori_loop(0, CHUNK, fill_one, 0)

    def zcopy(k, carry):
        r0 = rbase + k * ZR
        pltpu.async_copy(zrows, acc.at[pl.ds(r0, ZR)], zsem)
        pltpu.async_copy(z16, cnt.at[pl.ds(r0, ZR)], zsem)
        return carry

    lax.fori_loop(0, RPT // ZR, zcopy, 0)
    idx_in.wait()
    idx_in2.wait()

    def zdrain(k, carry):
        r0 = rbase + k * ZR
        pltpu.make_async_copy(zrows, acc.at[pl.ds(r0, ZR)], zsem).wait()
        pltpu.make_async_copy(z16, cnt.at[pl.ds(r0, ZR)], zsem).wait()
        return carry

    lax.fori_loop(0, RPT // ZR, zdrain, 0)
    plsc.subcore_barrier()

    my_x = xv_hbm.at[pl.ds(c, 2 * N - 1)]

    def start_g(j, b):
        pltpu.async_copy(my_x.at[sidx_all.at[j]], rows.at[b], gsem.at[b])

    def wait_g(j, b):
        pltpu.make_async_copy(
            my_x.at[sidx_all.at[j]], rows.at[b], gsem.at[b]).wait()

    def start_s(j, b):
        pltpu.async_copy(rows.at[b], acc.at[didx_all.at[j]], ssem.at[b],
                         add=True)

        @pl.when((j < HCHUNK) == (c == 0))
        def _():
            pltpu.async_copy(ones16, cnt.at[didx_all.at[j]], csem, add=True)

    def wait_s(j, b):
        pltpu.make_async_copy(
            rows.at[b], acc.at[didx_all.at[j]], ssem.at[b]).wait()

    for b in range(NB):
        start_g(b, b)

    def edge(k, carry):
        j = NB * k
        for b in range(NB):
            wait_g(j + b, b)
            start_s(j + b, b)
        for b in range(NB):
            wait_s(j + b, b)
            start_g(j + NB + b, b)
        return carry

    lax.fori_loop(0, NCHUNK // NB - 1, edge, 0)
    jlast = NCHUNK - NB
    for b in range(NB):
        wait_g(jlast + b, b)
        start_s(jlast + b, b)
    for b in range(NB):
        wait_s(jlast + b, b)

    def drain(k, carry):
        j = k + c * HCHUNK
        pltpu.make_async_copy(ones16, cnt.at[didx_all.at[j]], csem).wait()
        return carry

    lax.fori_loop(0, HCHUNK, drain, 0)

    plsc.subcore_barrier()

    OC = RPT // CHUNK

    def oc_in(k, b):
        pltpu.async_copy(acc.at[pl.ds(rbase + k * CHUNK, CHUNK)],
                         rows.at[b], gsem.at[b])

    def oc_wait(k, b):
        pltpu.make_async_copy(acc.at[pl.ds(rbase + k * CHUNK, CHUNK)],
                              rows.at[b], gsem.at[b]).wait()

    for b in range(NB):
        oc_in(b, b)
    for k in range(OC):
        b = k % NB
        oc_wait(k, b)
        pltpu.sync_copy(rows.at[b],
                        out_s.at[pl.ds(rbase + k * CHUNK, CHUNK),
                                 pl.ds(c * DH, DH)])
        if k + NB < OC:
            oc_in(k + NB, b)

    for k in range(RPT // ZR):
        r0 = rbase + k * ZR
        pltpu.sync_copy(cnt.at[pl.ds(r0, ZR)], z16)
        pltpu.sync_copy(z16, out_c.at[c, pl.ds(r0, ZR)])


_sc_segment_sum = pl.kernel(
    _sc_body,
    out_type=(
        jax.ShapeDtypeStruct((NP, D), jnp.float32),
        jax.ShapeDtypeStruct((NC, NP, 16), jnp.float32),
    ),
    mesh=plsc.VectorSubcoreMesh(
        core_axis_name="c", subcore_axis_name="s",
        num_cores=NC, num_subcores=NS),
    compiler_params=pltpu.CompilerParams(use_tc_tiling_on_sc=False),
    scratch_types=[
        pltpu.VMEM((NCHUNK, CHUNK), jnp.int32),
        pltpu.VMEM((NCHUNK, CHUNK), jnp.int32),
        pltpu.VMEM((NB, CHUNK, DH), jnp.float32),
        pltpu.VMEM((CHUNK, 16), jnp.float32),
        pltpu.VMEM((ZR, DH), jnp.float32),
        pltpu.VMEM((ZR, 16), jnp.float32),
        pltpu.SemaphoreType.DMA,
        pltpu.SemaphoreType.DMA((NB,)),
        pltpu.SemaphoreType.DMA((NB,)),
        pltpu.SemaphoreType.DMA,
        pltpu.SemaphoreType.DMA,
        pltpu.VMEM_SHARED((NP, DH), jnp.float32),
        pltpu.VMEM_SHARED((NP, 16), jnp.float32),
    ],
)


R = 1024
RB = R // 8


def _tc_body(s_ref, c0_ref, c1_ref, x_ref, wl_ref, wr_ref, b_ref, o_ref):
    ssum = s_ref[...]
    c8 = c0_ref[...] + c1_ref[...]
    rep = jnp.repeat(c8, 8, axis=0)
    p_row = lax.broadcasted_iota(jnp.int32, (R, D), 0)
    q_col = lax.broadcasted_iota(jnp.int32, (R, D), 1)
    sel = (q_col // 16) == (p_row % 8)
    cnt = jnp.sum(jnp.where(sel, rep, 0.0), axis=-1,
                  keepdims=True) * (1.0 / 16.0)
    mean = ssum / jnp.clip(cnt, 1.0, None)
    dn = (((1,), (1,)), ((), ()))
    out = (lax.dot_general(mean, wl_ref[...], dn,
                           preferred_element_type=jnp.float32)
           + lax.dot_general(x_ref[...], wr_ref[...], dn,
                             preferred_element_type=jnp.float32)
           + b_ref[...])
    nrm = jnp.sqrt(jnp.sum(out * out, axis=-1, keepdims=True))
    o_ref[...] = out / jnp.maximum(nrm, 1e-12)


_CB = NP // 8 // RB


def _tc_combine(ps, pcv, x, W_l, W_r, b2):
    return pl.pallas_call(
        _tc_body,
        grid=(N // R + 1,),
        in_specs=[
            pl.BlockSpec((R, D), lambda i: (i, 0)),
            pl.BlockSpec((RB, D), lambda i: (i, 0)),
            pl.BlockSpec((RB, D), lambda i: (i + _CB, 0)),
            pl.BlockSpec((R, D), lambda i: (i, 0)),
            pl.BlockSpec((D, D), lambda i: (0, 0)),
            pl.BlockSpec((D, D), lambda i: (0, 0)),
            pl.BlockSpec((1, D), lambda i: (0, 0)),
        ],
        out_specs=pl.BlockSpec((R, D), lambda i: (i, 0)),
        out_shape=jax.ShapeDtypeStruct((N, D), jnp.float32),
    )(ps, pcv, pcv, x, W_l, W_r, b2)


@jax.jit
def kernel(x, edge_index, W_l, W_r, b_l):
    src2 = (edge_index[0] * 2).reshape(E // CHUNK, CHUNK)
    dst2 = edge_index[1].reshape(E // CHUNK, CHUNK)
    xv = x.reshape(2 * N, DH)
    ps, pc = _sc_segment_sum(xv, src2, dst2)
    pcv = pc.reshape(NC * NP // 8, D)
    return _tc_combine(ps, pcv, x, W_l, W_r, b_l.reshape(1, D))


# speedup vs baseline: 1.0762x; 1.0762x over previous
import jax
import jax.numpy as jnp
from jax import lax
from jax.experimental import pallas as pl
from jax.experimental.pallas import tpu as pltpu
from jax.experimental.pallas import tpu_sc as plsc

N = 10000
E = 320000
D = 128
DH = D // 2

NC = 2
NS = 16
CPT = E // NS
CHUNK = 80
NCHUNK = CPT // CHUNK
HCHUNK = NCHUNK // 2
NP = 10240
RPT = NP // NS
ZR = 128


NB = 5


def _sc_body(xv_hbm, eidx_hbm, out_s, out_c,
             sidx_all, didx_all, rows, ones16, zrows, z16,
             isem, gsem, ssem, csem, zsem, acc, cnt):
    c = lax.axis_index("c")
    s = lax.axis_index("s")
    jbase = s * NCHUNK
    rbase = s * RPT

    zero16 = jnp.zeros((16,), jnp.float32)
    one16 = jnp.ones((16,), jnp.float32)

    idx_in = pltpu.async_copy(
        eidx_hbm.at[pl.ds(jbase, NCHUNK)], sidx_all, isem)
    idx_in2 = pltpu.async_copy(
        eidx_hbm.at[pl.ds(E // CHUNK + jbase, NCHUNK)], didx_all, isem)

    def fill_z(i, carry):
        for k in range(DH // 16):
            zrows[i, pl.ds(k * 16, 16)] = zero16
        z16[i, :] = zero16
        return carry

    lax.fori_loop(0, ZR, fill_z, 0)

    def fill_one(i, carry):
        ones16[i, :] = one16
        return carry

    lax.fori_loop(0, CHUNK, fill_one, 0)

    def zcopy(k, carry):
        r0 = rbase + k * ZR
        pltpu.async_copy(zrows, acc.at[pl.ds(r0, ZR)], zsem)
        pltpu.async_copy(z16, cnt.at[pl.ds(r0, ZR)], zsem)
        return carry

    lax.fori_loop(0, RPT // ZR, zcopy, 0)
    idx_in.wait()
    idx_in2.wait()

    cvec = jnp.full((16,), c, jnp.int32)

    def xform(r, carry):
        for g in range(CHUNK // 16):
            v = sidx_all[r, pl.ds(g * 16, 16)]
            sidx_all[r, pl.ds(g * 16, 16)] = v + v + cvec
        return carry

    lax.fori_loop(0, NCHUNK, xform, 0)

    def zdrain(k, carry):
        r0 = rbase + k * ZR
        pltpu.make_async_copy(zrows, acc.at[pl.ds(r0, ZR)], zsem).wait()
        pltpu.make_async_copy(z16, cnt.at[pl.ds(r0, ZR)], zsem).wait()
        return carry

    lax.fori_loop(0, RPT // ZR, zdrain, 0)
    plsc.subcore_barrier()

    def start_g(j, b):
        pltpu.async_copy(xv_hbm.at[sidx_all.at[j]], rows.at[b], gsem.at[b])

    def wait_g(j, b):
        pltpu.make_async_copy(
            xv_hbm.at[sidx_all.at[j]], rows.at[b], gsem.at[b]).wait()

    def start_s(j, b):
        pltpu.async_copy(rows.at[b], acc.at[didx_all.at[j]], ssem.at[b],
                         add=True)

        @pl.when((j < HCHUNK) == (c == 0))
        def _():
            pltpu.async_copy(ones16, cnt.at[didx_all.at[j]], csem, add=True)

    def wait_s(j, b):
        pltpu.make_async_copy(
            rows.at[b], acc.at[didx_all.at[j]], ssem.at[b]).wait()

    for b in range(NB):
        start_g(b, b)

    def edge(k, carry):
        j = NB * k
        for b in range(NB):
            wait_g(j + b, b)
            start_s(j + b, b)
        for b in range(NB):
            wait_s(j + b, b)
            start_g(j + NB + b, b)
        return carry

    lax.fori_loop(0, NCHUNK // NB - 1, edge, 0)
    jlast = NCHUNK - NB
    for b in range(NB):
        wait_g(jlast + b, b)
        start_s(jlast + b, b)
    for b in range(NB):
        wait_s(jlast + b, b)

    def drain(k, carry):
        j = k + c * HCHUNK
        pltpu.make_async_copy(ones16, cnt.at[didx_all.at[j]], csem).wait()
        return carry

    lax.fori_loop(0, HCHUNK, drain, 0)

    plsc.subcore_barrier()

    OC = RPT // CHUNK

    def oc_in(k, b):
        pltpu.async_copy(acc.at[pl.ds(rbase + k * CHUNK, CHUNK)],
                         rows.at[b], gsem.at[b])

    def oc_wait(k, b):
        pltpu.make_async_copy(acc.at[pl.ds(rbase + k * CHUNK, CHUNK)],
                              rows.at[b], gsem.at[b]).wait()

    for b in range(NB):
        oc_in(b, b)
    for k in range(OC):
        b = k % NB
        oc_wait(k, b)
        pltpu.sync_copy(rows.at[b],
                        out_s.at[pl.ds(rbase + k * CHUNK, CHUNK),
                                 pl.ds(c * DH, DH)])
        if k + NB < OC:
            oc_in(k + NB, b)

    for k in range(RPT // ZR):
        r0 = rbase + k * ZR
        pltpu.sync_copy(cnt.at[pl.ds(r0, ZR)], z16)
        pltpu.sync_copy(z16, out_c.at[c, pl.ds(r0, ZR)])


_sc_segment_sum = pl.kernel(
    _sc_body,
    out_type=(
        jax.ShapeDtypeStruct((NP, D), jnp.float32),
        jax.ShapeDtypeStruct((NC, NP, 16), jnp.float32),
    ),
    mesh=plsc.VectorSubcoreMesh(
        core_axis_name="c", subcore_axis_name="s",
        num_cores=NC, num_subcores=NS),
    compiler_params=pltpu.CompilerParams(use_tc_tiling_on_sc=False),
    scratch_types=[
        pltpu.VMEM((NCHUNK, CHUNK), jnp.int32),
        pltpu.VMEM((NCHUNK, CHUNK), jnp.int32),
        pltpu.VMEM((NB, CHUNK, DH), jnp.float32),
        pltpu.VMEM((CHUNK, 16), jnp.float32),
        pltpu.VMEM((ZR, DH), jnp.float32),
        pltpu.VMEM((ZR, 16), jnp.float32),
        pltpu.SemaphoreType.DMA,
        pltpu.SemaphoreType.DMA((NB,)),
        pltpu.SemaphoreType.DMA((NB,)),
        pltpu.SemaphoreType.DMA,
        pltpu.SemaphoreType.DMA,
        pltpu.VMEM_SHARED((NP, DH), jnp.float32),
        pltpu.VMEM_SHARED((NP, 16), jnp.float32),
    ],
)


R = 1024
RB = R // 8


def _tc_body(s_ref, c0_ref, c1_ref, x_ref, wl_ref, wr_ref, b_ref, o_ref):
    ssum = s_ref[...]
    c8 = c0_ref[...] + c1_ref[...]
    rep = jnp.repeat(c8, 8, axis=0)
    p_row = lax.broadcasted_iota(jnp.int32, (R, D), 0)
    q_col = lax.broadcasted_iota(jnp.int32, (R, D), 1)
    sel = (q_col // 16) == (p_row % 8)
    cnt = jnp.sum(jnp.where(sel, rep, 0.0), axis=-1,
                  keepdims=True) * (1.0 / 16.0)
    mean = ssum / jnp.clip(cnt, 1.0, None)
    dn = (((1,), (1,)), ((), ()))
    out = (lax.dot_general(mean, wl_ref[...], dn,
                           preferred_element_type=jnp.float32)
           + lax.dot_general(x_ref[...], wr_ref[...], dn,
                             preferred_element_type=jnp.float32)
           + b_ref[...])
    nrm = jnp.sqrt(jnp.sum(out * out, axis=-1, keepdims=True))
    o_ref[...] = out / jnp.maximum(nrm, 1e-12)


_CB = NP // 8 // RB


def _tc_combine(ps, pcv, x, W_l, W_r, b2):
    return pl.pallas_call(
        _tc_body,
        grid=(N // R + 1,),
        in_specs=[
            pl.BlockSpec((R, D), lambda i: (i, 0)),
            pl.BlockSpec((RB, D), lambda i: (i, 0)),
            pl.BlockSpec((RB, D), lambda i: (i + _CB, 0)),
            pl.BlockSpec((R, D), lambda i: (i, 0)),
            pl.BlockSpec((D, D), lambda i: (0, 0)),
            pl.BlockSpec((D, D), lambda i: (0, 0)),
            pl.BlockSpec((1, D), lambda i: (0, 0)),
        ],
        out_specs=pl.BlockSpec((R, D), lambda i: (i, 0)),
        out_shape=jax.ShapeDtypeStruct((N, D), jnp.float32),
    )(ps, pcv, pcv, x, W_l, W_r, b2)


@jax.jit
def kernel(x, edge_index, W_l, W_r, b_l):
    e2 = edge_index.reshape(2 * (E // CHUNK), CHUNK)
    xv = x.reshape(2 * N, DH)
    ps, pc = _sc_segment_sum(xv, e2)
    pcv = pc.reshape(NC * NP // 8, D)
    return _tc_combine(ps, pcv, x, W_l, W_r, b_l.reshape(1, D))
